# Initial kernel scaffold; baseline (speedup 1.0000x reference)
import jax
import jax.numpy as jnp
from jax.experimental import pallas as pl


def kernel(x, mask, edge_index, s1_w0, s1_b0, s1_wg, s1_bg, s1_wb, s1_bb, s2_w0, s2_b0, s2_wg, s2_bg, s2_wb, s2_bb, ss_w0, ss_b0, ss_wg, ss_bg, ss_wb, ss_bb, g1_w, g1_as, g1_ad, g1_b, g2_w, g2_as, g2_ad, g2_b, gs_w, gs_as, gs_ad, gs_b):
    raise NotImplementedError("write your pallas kernel here")


# speedup vs baseline: 39.7524x; 39.7524x over previous
import functools
import jax
import jax.numpy as jnp
from jax import lax
from jax.experimental import pallas as pl
from jax.experimental.pallas import tpu as pltpu
from jax.experimental.pallas import tpu_sc as plsc

N = 10000
E = 320000
C = 128
CM = 16
HID = 128
H = 4
CPH = 32
EPS = 1e-05

NC = 2
NS = 16
NW = NC * NS
EPW = E // NW
CH = 200
NCH = EPW // CH
NP = 10240
RP = NP // NS
BR = 2000
NG = N // BR

_MESH = dict(
    mesh=plsc.VectorSubcoreMesh(core_axis_name="c", subcore_axis_name="s"),
    compiler_params=pltpu.CompilerParams(use_tc_tiling_on_sc=False,
                                         needs_layout_passes=False),
)
_F32 = jnp.float32


def _wid():
    return lax.axis_index("s") * NC + lax.axis_index("c")


def _sid():
    return lax.axis_index("s")


def _sc_deg(dst, z16, ones16):
    def body(dst_h, z_h, ones_h, out_h, idx_v, ones_v, acc_sh):
        c = lax.axis_index("c")
        s = _sid()
        wid = _wid()
        pltpu.sync_copy(z_h.at[pl.ds(s * RP, RP)], acc_sh.at[pl.ds(s * RP, RP)])
        pltpu.sync_copy(ones_h, ones_v)
        plsc.subcore_barrier()

        def step(j, carry):
            base = wid * EPW + j * CH
            pltpu.sync_copy(dst_h.at[pl.ds(base, CH)], idx_v)
            pltpu.sync_copy(ones_v, acc_sh.at[idx_v], add=True)
            return carry

        lax.fori_loop(0, NCH, step, 0)
        plsc.subcore_barrier()
        pltpu.sync_copy(acc_sh.at[pl.ds(s * RP, RP)],
                        out_h.at[pl.ds(c * NP + s * RP, RP)])

    f = pl.kernel(
        body,
        out_type=jax.ShapeDtypeStruct((NC * NP, 16), _F32),
        scratch_types=[
            pltpu.VMEM((CH,), jnp.int32),
            pltpu.VMEM((CH, 16), _F32),
            pltpu.VMEM_SHARED((NP, 16), _F32),
        ],
        **_MESH,
    )
    return f(dst, z16, ones16)


def _sc_seg(tabs, src, dst, zk, k):
    P = len(tabs)

    def body(*refs):
        tab_h = refs[:P]
        src_h, dst_h, z_h = refs[P:P + 3]
        out_h = refs[P + 3:P + 3 + P]
        idx_s, idx_d, rows_v, acc_sh, sem = refs[P + 3 + P:]
        c = lax.axis_index("c")
        s = _sid()
        wid = _wid()
        for p in range(P):
            pltpu.sync_copy(z_h.at[pl.ds(s * RP, RP)], acc_sh.at[pl.ds(s * RP, RP)])
            plsc.subcore_barrier()

            def step(j, carry, p=p):
                base = wid * EPW + j * CH
                pltpu.sync_copy(src_h.at[pl.ds(base, CH)], idx_s)
                pltpu.sync_copy(dst_h.at[pl.ds(base, CH)], idx_d)
                pltpu.async_copy(tab_h[p].at[idx_s], rows_v, sem).wait()
                pltpu.sync_copy(rows_v, acc_sh.at[idx_d], add=True)
                return carry

            lax.fori_loop(0, NCH, step, 0)
            plsc.subcore_barrier()
            pltpu.sync_copy(acc_sh.at[pl.ds(s * RP, RP)],
                            out_h[p].at[pl.ds(c * NP + s * RP, RP)])
            plsc.subcore_barrier()

    f = pl.kernel(
        body,
        out_type=[jax.ShapeDtypeStruct((NC * NP, k), _F32) for _ in range(P)],
        scratch_types=[
            pltpu.VMEM((CH,), jnp.int32),
            pltpu.VMEM((CH,), jnp.int32),
            pltpu.VMEM((CH, k), _F32),
            pltpu.VMEM_SHARED((NP, k), _F32),
            pltpu.SemaphoreType.DMA,
        ],
        **_MESH,
    )
    return f(*tabs, src, dst, zk)


def _sc_gat(gats, src, dst, z128, z16):
    G = len(gats)

    def body(*refs):
        ins = refs[:4 * G]
        src_h, dst_h, z128_h, z16_h = refs[4 * G:4 * G + 4]
        outs = refs[4 * G + 4:4 * G + 4 + 2 * G]
        (idx_s, idx_d, rows_v, esr, edr, exb, m_v, acc_sh, den_sh, sem) = \
            refs[4 * G + 4 + 2 * G:]
        c = lax.axis_index("c")
        s = _sid()
        wid = _wid()
        for g in range(G):
            h_h, es_h, ed_h, m_h = ins[4 * g:4 * g + 4]
            acc_o, den_o = outs[2 * g:2 * g + 2]
            pltpu.sync_copy(z128_h.at[pl.ds(s * RP, RP)],
                            acc_sh.at[pl.ds(s * RP, RP)])
            pltpu.sync_copy(z16_h.at[pl.ds(s * RP, RP)],
                            den_sh.at[pl.ds(s * RP, RP)])
            pltpu.sync_copy(m_h, m_v)
            plsc.subcore_barrier()

            def step(j, carry, h_h=h_h, es_h=es_h, ed_h=ed_h):
                base = wid * EPW + j * CH
                pltpu.sync_copy(src_h.at[pl.ds(base, CH)], idx_s)
                pltpu.sync_copy(dst_h.at[pl.ds(base, CH)], idx_d)
                d1 = pltpu.async_copy(h_h.at[idx_s], rows_v, sem)
                d2 = pltpu.async_copy(es_h.at[idx_s], esr, sem)
                d3 = pltpu.async_copy(ed_h.at[idx_d], edr, sem)
                d1.wait()
                d2.wait()
                d3.wait()
                mv = m_v[0, :]

                def row(r, rcarry):
                    e16 = esr[r, :] + edr[r, :]
                    e16 = jnp.maximum(e16, e16 * 0.2)
                    ex16 = jnp.exp(e16 - mv)
                    exb[r, :] = ex16
                    for gi in range(8):
                        spl = plsc.load_gather(
                            exb,
                            [jnp.full((16,), r, jnp.int32),
                             jnp.full((16,), gi // 2, jnp.int32)])
                        rows_v[r, pl.ds(gi * 16, 16)] = \
                            rows_v[r, pl.ds(gi * 16, 16)] * spl
                    return rcarry

                lax.fori_loop(0, CH, row, 0)
                pltpu.sync_copy(rows_v, acc_sh.at[idx_d], add=True)
                pltpu.sync_copy(exb, den_sh.at[idx_d], add=True)
                return carry

            lax.fori_loop(0, NCH, step, 0)
            plsc.subcore_barrier()
            pltpu.sync_copy(acc_sh.at[pl.ds(s * RP, RP)],
                            acc_o.at[pl.ds(c * NP + s * RP, RP)])
            pltpu.sync_copy(den_sh.at[pl.ds(s * RP, RP)],
                            den_o.at[pl.ds(c * NP + s * RP, RP)])
            plsc.subcore_barrier()

    out_type = []
    for _ in range(G):
        out_type.append(jax.ShapeDtypeStruct((NC * NP, 128), _F32))
        out_type.append(jax.ShapeDtypeStruct((NC * NP, 16), _F32))
    f = pl.kernel(
        body,
        out_type=out_type,
        scratch_types=[
            pltpu.VMEM((CH,), jnp.int32),
            pltpu.VMEM((CH,), jnp.int32),
            pltpu.VMEM((CH, 128), _F32),
            pltpu.VMEM((CH, 16), _F32),
            pltpu.VMEM((CH, 16), _F32),
            pltpu.VMEM((CH, 16), _F32),
            pltpu.VMEM((1, 16), _F32),
            pltpu.VMEM_SHARED((NP, 128), _F32),
            pltpu.VMEM_SHARED((NP, 16), _F32),
            pltpu.SemaphoreType.DMA,
        ],
        **_MESH,
    )
    flat = []
    for g in gats:
        flat.extend(g)
    return f(*flat, src, dst, z128, z16)


def _k1_body(d0, d1, mask_r, x_r, dinv_o, ym_o, st_o):
    deg = d0[:, 0:1] + d1[:, 0:1] + 1.0
    dinv = lax.rsqrt(deg)
    dinv_o[...] = dinv
    ym_o[...] = mask_r[...] * dinv
    xv = x_r[...]
    mu = jnp.mean(xv, axis=0, keepdims=True)
    var = jnp.mean(xv * xv, axis=0, keepdims=True) - mu * mu
    st_o[...] = jnp.concatenate([mu, lax.rsqrt(var + EPS)], axis=0)


def _k1(degacc, mask, x):
    return pl.pallas_call(
        _k1_body,
        out_shape=[
            jax.ShapeDtypeStruct((N, 1), _F32),
            jax.ShapeDtypeStruct((N, 16), _F32),
            jax.ShapeDtypeStruct((2, 128), _F32),
        ],
    )(degacc[:N], degacc[NP:NP + N], mask, x)


def _k2_body(s0, s1, ym_r, dinv_r, w_r, b_r, y_o):
    dinv = dinv_r[...]
    S = dinv * (s0[...] + s1[...] + ym_r[...])
    for p in range(3):
        mf = jnp.dot(S, w_r[p], preferred_element_type=_F32) + b_r[p]
        mf = jnp.maximum(mf, 0.0)
        y_o[p] = mf * dinv


def _k2(sraw, ym, dinv, w0s, b0s):
    return pl.pallas_call(
        _k2_body,
        grid=(NG,),
        in_specs=[
            pl.BlockSpec((BR, 16), lambda i: (i, 0)),
            pl.BlockSpec((BR, 16), lambda i: (i, 0)),
            pl.BlockSpec((BR, 16), lambda i: (i, 0)),
            pl.BlockSpec((BR, 1), lambda i: (i, 0)),
            pl.BlockSpec((3, CM, HID), lambda i: (0, 0, 0)),
            pl.BlockSpec((3, HID), lambda i: (0, 0)),
        ],
        out_specs=pl.BlockSpec((3, BR, HID), lambda i: (0, i, 0)),
        out_shape=jax.ShapeDtypeStruct((3, N, HID), _F32),
    )(sraw[:N], sraw[NP:NP + N], ym, dinv, w0s, b0s)


def _k3_body(t0, t1, y_r, dinv_r, wg, bg, wb, bb, xin, st, W, a_s, a_d,
             h_o, es_o, ed_o, m_o):
    i = pl.program_id(0)
    dinv = dinv_r[...]
    T = dinv * (t0[...] + t1[...] + y_r[...])
    gamma = jnp.dot(T, wg[...], preferred_element_type=_F32) + bg[...]
    beta = jnp.dot(T, wb[...], preferred_element_type=_F32) + bb[...]
    xb = (xin[...] - st[0:1, :]) * st[1:2, :]
    xa = xb * (1.0 + gamma) + beta
    xa = jnp.maximum(xa, 0.2 * xa)
    h = jnp.dot(xa, W[...], preferred_element_type=_F32)
    h_o[...] = h
    h4 = h.reshape(BR, H, CPH)
    es = jnp.sum(h4 * a_s[...][None], axis=2)
    ed = jnp.sum(h4 * a_d[...][None], axis=2)
    zpad = jnp.zeros((BR, 12), _F32)
    es_o[...] = jnp.concatenate([es, zpad], axis=1)
    ed_o[...] = jnp.concatenate([ed, zpad], axis=1)
    cur = jnp.concatenate(
        [jnp.max(es, axis=0), jnp.max(ed, axis=0),
         jnp.full((8,), -1e30, _F32)]).reshape(1, 16)

    @pl.when(i == 0)
    def _():
        m_o[...] = cur

    @pl.when(i > 0)
    def _():
        m_o[...] = jnp.maximum(m_o[...], cur)

    @pl.when(i == NG - 1)
    def _():
        acc = m_o[...]
        mx = acc[0, :4] + acc[0, 4:8]
        M = jnp.maximum(mx, 0.2 * mx)
        m_o[...] = jnp.tile(M, 4).reshape(1, 16)


def _k3(traw, y, dinv, wg, bg, wb, bb, xin, st, W, a_s, a_d):
    blk128 = pl.BlockSpec((BR, C), lambda i: (i, 0))
    return pl.pallas_call(
        _k3_body,
        grid=(NG,),
        in_specs=[
            blk128, blk128, blk128,
            pl.BlockSpec((BR, 1), lambda i: (i, 0)),
            pl.BlockSpec((HID, C), lambda i: (0, 0)),
            pl.BlockSpec((C,), lambda i: (0,)),
            pl.BlockSpec((HID, C), lambda i: (0, 0)),
            pl.BlockSpec((C,), lambda i: (0,)),
            blk128,
            pl.BlockSpec((2, 128), lambda i: (0, 0)),
            pl.BlockSpec((C, C), lambda i: (0, 0)),
            pl.BlockSpec((H, CPH), lambda i: (0, 0)),
            pl.BlockSpec((H, CPH), lambda i: (0, 0)),
        ],
        out_specs=[
            blk128,
            pl.BlockSpec((BR, 16), lambda i: (i, 0)),
            pl.BlockSpec((BR, 16), lambda i: (i, 0)),
            pl.BlockSpec((1, 16), lambda i: (0, 0)),
        ],
        out_shape=[
            jax.ShapeDtypeStruct((N, C), _F32),
            jax.ShapeDtypeStruct((N, 16), _F32),
            jax.ShapeDtypeStruct((N, 16), _F32),
            jax.ShapeDtypeStruct((1, 16), _F32),
        ],
    )(traw[:N], traw[NP:NP + N], y, dinv, wg, bg, wb, bb, xin, st, W, a_s, a_d)


def _k4_body(a0, a1, d0, d1, h_r, es_r, ed_r, m_r, b_r, add_r, out_o, st_o):
    i = pl.program_id(0)
    e = es_r[...] + ed_r[...]
    e = jnp.maximum(e, 0.2 * e)
    exl = jnp.exp(e - m_r[...])
    den = d0[...] + d1[...] + exl
    h = h_r[...]
    num = a0[...] + a1[...] + (h.reshape(BR, H, CPH)
                               * exl[:, :H, None]).reshape(BR, C)
    out = (num.reshape(BR, H, CPH) / den[:, :H, None]).reshape(BR, C)
    out = out + b_r[...] + add_r[...]
    out_o[...] = out
    cur = jnp.concatenate([jnp.sum(out, axis=0, keepdims=True),
                           jnp.sum(out * out, axis=0, keepdims=True)], axis=0)

    @pl.when(i == 0)
    def _():
        st_o[...] = cur

    @pl.when(i > 0)
    def _():
        st_o[...] = st_o[...] + cur

    @pl.when(i == NG - 1)
    def _():
        acc = st_o[...]
        mu = acc[0:1, :] / N
        var = acc[1:2, :] / N - mu * mu
        st_o[...] = jnp.concatenate([mu, lax.rsqrt(var + EPS)], axis=0)


def _k4(acc, den, h, es, ed, m16, b, add):
    blk128 = pl.BlockSpec((BR, C), lambda i: (i, 0))
    blk16 = pl.BlockSpec((BR, 16), lambda i: (i, 0))
    return pl.pallas_call(
        _k4_body,
        grid=(NG,),
        in_specs=[
            blk128, blk128, blk16, blk16, blk128, blk16, blk16,
            pl.BlockSpec((1, 16), lambda i: (0, 0)),
            pl.BlockSpec((C,), lambda i: (0,)),
            blk128,
        ],
        out_specs=[
            blk128,
            pl.BlockSpec((2, 128), lambda i: (0, 0)),
        ],
        out_shape=[
            jax.ShapeDtypeStruct((N, C), _F32),
            jax.ShapeDtypeStruct((2, 128), _F32),
        ],
    )(acc[:N], acc[NP:NP + N], den[:N], den[NP:NP + N], h, es, ed, m16, b, add)


@jax.jit
def kernel(x, mask, edge_index,
           s1_w0, s1_b0, s1_wg, s1_bg, s1_wb, s1_bb,
           s2_w0, s2_b0, s2_wg, s2_bg, s2_wb, s2_bb,
           ss_w0, ss_b0, ss_wg, ss_bg, ss_wb, ss_bb,
           g1_w, g1_as, g1_ad, g1_b,
           g2_w, g2_as, g2_ad, g2_b,
           gs_w, gs_as, gs_ad, gs_b):
    src = edge_index[0].astype(jnp.int32)
    dst = edge_index[1].astype(jnp.int32)
    z16 = jnp.zeros((NP, 16), _F32)
    z128 = jnp.zeros((NP, 128), _F32)
    ones16 = jnp.ones((CH, 16), _F32)
    zN = jnp.zeros((N, C), _F32)

    degacc = _sc_deg(dst, z16, ones16)
    dinv, ym, xst = _k1(degacc, mask, x)
    (sraw,) = _sc_seg([ym], src, dst, z16, 16)
    w0s = jnp.stack([s1_w0, s2_w0, ss_w0])
    b0s = jnp.stack([s1_b0, s2_b0, ss_b0])
    Y = _k2(sraw, ym, dinv, w0s, b0s)
    t1, t2, ts = _sc_seg([Y[0], Y[1], Y[2]], src, dst, z128, 128)

    h1, es1, ed1, m1 = _k3(t1, Y[0], dinv, s1_wg, s1_bg, s1_wb, s1_bb,
                           x, xst, g1_w, g1_as, g1_ad)
    hs, ess, eds, ms = _k3(ts, Y[2], dinv, ss_wg, ss_bg, ss_wb, ss_bb,
                           x, xst, gs_w, gs_as, gs_ad)
    acc1, den1, accs, dens = _sc_gat(
        [(h1, es1, ed1, m1), (hs, ess, eds, ms)], src, dst, z128, z16)
    x1, x1st = _k4(acc1, den1, h1, es1, ed1, m1, g1_b, zN)
    xs, _ = _k4(accs, dens, hs, ess, eds, ms, gs_b, zN)

    h2, es2, ed2, m2 = _k3(t2, Y[1], dinv, s2_wg, s2_bg, s2_wb, s2_bb,
                           x1, x1st, g2_w, g2_as, g2_ad)
    acc2, den2 = _sc_gat([(h2, es2, ed2, m2)], src, dst, z128, z16)
    out, _ = _k4(acc2, den2, h2, es2, ed2, m2, g2_b, xs)
    return out


# speedup vs baseline: 69.7028x; 1.7534x over previous
import functools
import jax
import jax.numpy as jnp
from jax import lax
from jax.experimental import pallas as pl
from jax.experimental.pallas import tpu as pltpu
from jax.experimental.pallas import tpu_sc as plsc

N = 10000
E = 320000
C = 128
CM = 16
HID = 128
H = 4
CPH = 32
EPS = 1e-05

NC = 2
NS = 16
NW = NC * NS
EPW = E // NW
CH = 200
NCH = EPW // CH
NP = 10240
RP = NP // NS
BR = 2000
NG = N // BR

_MESH = dict(
    mesh=plsc.VectorSubcoreMesh(core_axis_name="c", subcore_axis_name="s"),
    compiler_params=pltpu.CompilerParams(use_tc_tiling_on_sc=False,
                                         needs_layout_passes=False),
)
_F32 = jnp.float32


def _wid():
    return lax.axis_index("s") * NC + lax.axis_index("c")


def _sid():
    return lax.axis_index("s")


def _sc_deg(dst, z16, ones16):
    def body(dst_h, z_h, ones_h, out_h, idx_v, ones_v, acc_sh):
        c = lax.axis_index("c")
        s = _sid()
        wid = _wid()
        pltpu.sync_copy(z_h.at[pl.ds(s * RP, RP)], acc_sh.at[pl.ds(s * RP, RP)])
        pltpu.sync_copy(ones_h, ones_v)
        plsc.subcore_barrier()

        def step(j, carry):
            base = wid * EPW + j * CH
            pltpu.sync_copy(dst_h.at[pl.ds(base, CH)], idx_v)
            pltpu.sync_copy(ones_v, acc_sh.at[idx_v], add=True)
            return carry

        lax.fori_loop(0, NCH, step, 0)
        plsc.subcore_barrier()
        pltpu.sync_copy(acc_sh.at[pl.ds(s * RP, RP)],
                        out_h.at[pl.ds(c * NP + s * RP, RP)])

    f = pl.kernel(
        body,
        out_type=jax.ShapeDtypeStruct((NC * NP, 16), _F32),
        scratch_types=[
            pltpu.VMEM((CH,), jnp.int32),
            pltpu.VMEM((CH, 16), _F32),
            pltpu.VMEM_SHARED((NP, 16), _F32),
        ],
        **_MESH,
    )
    return f(dst, z16, ones16)


def _sc_seg(tabs, src, dst, zk, k):
    P = len(tabs)

    def body(*refs):
        tab_h = refs[:P]
        src_h, dst_h, z_h = refs[P:P + 3]
        out_h = refs[P + 3:P + 3 + P]
        idx_s, idx_d, rows_v, acc_sh, sem = refs[P + 3 + P:]
        c = lax.axis_index("c")
        s = _sid()
        wid = _wid()
        for p in range(P):
            pltpu.sync_copy(z_h.at[pl.ds(s * RP, RP)], acc_sh.at[pl.ds(s * RP, RP)])
            plsc.subcore_barrier()

            def step(j, carry, p=p):
                base = wid * EPW + j * CH
                pltpu.sync_copy(src_h.at[pl.ds(base, CH)], idx_s)
                pltpu.sync_copy(dst_h.at[pl.ds(base, CH)], idx_d)
                pltpu.async_copy(tab_h[p].at[idx_s], rows_v, sem).wait()
                pltpu.sync_copy(rows_v, acc_sh.at[idx_d], add=True)
                return carry

            lax.fori_loop(0, NCH, step, 0)
            plsc.subcore_barrier()
            pltpu.sync_copy(acc_sh.at[pl.ds(s * RP, RP)],
                            out_h[p].at[pl.ds(c * NP + s * RP, RP)])
            plsc.subcore_barrier()

    f = pl.kernel(
        body,
        out_type=[jax.ShapeDtypeStruct((NC * NP, k), _F32) for _ in range(P)],
        scratch_types=[
            pltpu.VMEM((CH,), jnp.int32),
            pltpu.VMEM((CH,), jnp.int32),
            pltpu.VMEM((CH, k), _F32),
            pltpu.VMEM_SHARED((NP, k), _F32),
            pltpu.SemaphoreType.DMA,
        ],
        **_MESH,
    )
    return f(*tabs, src, dst, zk)


def _sc_gat(gats, src, dst, z128, z16):
    G = len(gats)

    def body(*refs):
        ins = refs[:4 * G]
        src_h, dst_h, z128_h, z16_h = refs[4 * G:4 * G + 4]
        outs = refs[4 * G + 4:4 * G + 4 + 2 * G]
        (idx_s, idx_d, rows_v, esr, edr, exb, m_v, acc_sh, den_sh, sem) = \
            refs[4 * G + 4 + 2 * G:]
        c = lax.axis_index("c")
        s = _sid()
        wid = _wid()
        for g in range(G):
            h_h, es_h, ed_h, m_h = ins[4 * g:4 * g + 4]
            acc_o, den_o = outs[2 * g:2 * g + 2]
            pltpu.sync_copy(z128_h.at[pl.ds(s * RP, RP)],
                            acc_sh.at[pl.ds(s * RP, RP)])
            pltpu.sync_copy(z16_h.at[pl.ds(s * RP, RP)],
                            den_sh.at[pl.ds(s * RP, RP)])
            pltpu.sync_copy(m_h, m_v)
            plsc.subcore_barrier()

            def step(j, carry, h_h=h_h, es_h=es_h, ed_h=ed_h):
                base = wid * EPW + j * CH
                pltpu.sync_copy(src_h.at[pl.ds(base, CH)], idx_s)
                pltpu.sync_copy(dst_h.at[pl.ds(base, CH)], idx_d)
                d1 = pltpu.async_copy(h_h.at[idx_s], rows_v, sem)
                d2 = pltpu.async_copy(es_h.at[idx_s], esr, sem)
                d3 = pltpu.async_copy(ed_h.at[idx_d], edr, sem)
                d1.wait()
                d2.wait()
                d3.wait()
                mv = m_v[0, :]

                @plsc.parallel_loop(0, CH, unroll=4)
                def row(r):
                    e16 = esr[r, :] + edr[r, :]
                    e16 = jnp.maximum(e16, e16 * 0.2)
                    ex16 = jnp.exp(e16 - mv)
                    exb[r, :] = ex16
                    for hh in range(4):
                        spl = ex16.at[jnp.full((16,), hh, jnp.int32)].get(
                            mode="promise_in_bounds")
                        rows_v[r, pl.ds(hh * 32, 16)] = \
                            rows_v[r, pl.ds(hh * 32, 16)] * spl
                        rows_v[r, pl.ds(hh * 32 + 16, 16)] = \
                            rows_v[r, pl.ds(hh * 32 + 16, 16)] * spl
                pltpu.sync_copy(rows_v, acc_sh.at[idx_d], add=True)
                pltpu.sync_copy(exb, den_sh.at[idx_d], add=True)
                return carry

            lax.fori_loop(0, NCH, step, 0)
            plsc.subcore_barrier()
            pltpu.sync_copy(acc_sh.at[pl.ds(s * RP, RP)],
                            acc_o.at[pl.ds(c * NP + s * RP, RP)])
            pltpu.sync_copy(den_sh.at[pl.ds(s * RP, RP)],
                            den_o.at[pl.ds(c * NP + s * RP, RP)])
            plsc.subcore_barrier()

    out_type = []
    for _ in range(G):
        out_type.append(jax.ShapeDtypeStruct((NC * NP, 128), _F32))
        out_type.append(jax.ShapeDtypeStruct((NC * NP, 16), _F32))
    f = pl.kernel(
        body,
        out_type=out_type,
        scratch_types=[
            pltpu.VMEM((CH,), jnp.int32),
            pltpu.VMEM((CH,), jnp.int32),
            pltpu.VMEM((CH, 128), _F32),
            pltpu.VMEM((CH, 16), _F32),
            pltpu.VMEM((CH, 16), _F32),
            pltpu.VMEM((CH, 16), _F32),
            pltpu.VMEM((1, 16), _F32),
            pltpu.VMEM_SHARED((NP, 128), _F32),
            pltpu.VMEM_SHARED((NP, 16), _F32),
            pltpu.SemaphoreType.DMA,
        ],
        **_MESH,
    )
    flat = []
    for g in gats:
        flat.extend(g)
    return f(*flat, src, dst, z128, z16)


def _k1_body(d0, d1, mask_r, x_r, dinv_o, ym_o, st_o):
    deg = d0[:, 0:1] + d1[:, 0:1] + 1.0
    dinv = lax.rsqrt(deg)
    dinv_o[...] = dinv
    ym_o[...] = mask_r[...] * dinv
    xv = x_r[...]
    mu = jnp.mean(xv, axis=0, keepdims=True)
    var = jnp.mean(xv * xv, axis=0, keepdims=True) - mu * mu
    st_o[...] = jnp.concatenate([mu, lax.rsqrt(var + EPS)], axis=0)


def _k1(degacc, mask, x):
    return pl.pallas_call(
        _k1_body,
        out_shape=[
            jax.ShapeDtypeStruct((N, 1), _F32),
            jax.ShapeDtypeStruct((N, 16), _F32),
            jax.ShapeDtypeStruct((2, 128), _F32),
        ],
    )(degacc[:N], degacc[NP:NP + N], mask, x)


def _k2_body(s0, s1, ym_r, dinv_r, w_r, b_r, y_o):
    dinv = dinv_r[...]
    S = dinv * (s0[...] + s1[...] + ym_r[...])
    for p in range(3):
        mf = jnp.dot(S, w_r[p], preferred_element_type=_F32) + b_r[p]
        mf = jnp.maximum(mf, 0.0)
        y_o[p] = mf * dinv


def _k2(sraw, ym, dinv, w0s, b0s):
    return pl.pallas_call(
        _k2_body,
        grid=(NG,),
        in_specs=[
            pl.BlockSpec((BR, 16), lambda i: (i, 0)),
            pl.BlockSpec((BR, 16), lambda i: (i, 0)),
            pl.BlockSpec((BR, 16), lambda i: (i, 0)),
            pl.BlockSpec((BR, 1), lambda i: (i, 0)),
            pl.BlockSpec((3, CM, HID), lambda i: (0, 0, 0)),
            pl.BlockSpec((3, HID), lambda i: (0, 0)),
        ],
        out_specs=pl.BlockSpec((3, BR, HID), lambda i: (0, i, 0)),
        out_shape=jax.ShapeDtypeStruct((3, N, HID), _F32),
    )(sraw[:N], sraw[NP:NP + N], ym, dinv, w0s, b0s)


def _k3_body(t0, t1, y_r, dinv_r, wg, bg, wb, bb, xin, st, W, a_s, a_d,
             h_o, es_o, ed_o, m_o):
    i = pl.program_id(0)
    dinv = dinv_r[...]
    T = dinv * (t0[...] + t1[...] + y_r[...])
    gamma = jnp.dot(T, wg[...], preferred_element_type=_F32) + bg[...]
    beta = jnp.dot(T, wb[...], preferred_element_type=_F32) + bb[...]
    xb = (xin[...] - st[0:1, :]) * st[1:2, :]
    xa = xb * (1.0 + gamma) + beta
    xa = jnp.maximum(xa, 0.2 * xa)
    h = jnp.dot(xa, W[...], preferred_element_type=_F32)
    h_o[...] = h
    h4 = h.reshape(BR, H, CPH)
    es = jnp.sum(h4 * a_s[...][None], axis=2)
    ed = jnp.sum(h4 * a_d[...][None], axis=2)
    zpad = jnp.zeros((BR, 12), _F32)
    es_o[...] = jnp.concatenate([es, zpad], axis=1)
    ed_o[...] = jnp.concatenate([ed, zpad], axis=1)
    cur = jnp.concatenate(
        [jnp.max(es, axis=0), jnp.max(ed, axis=0),
         jnp.full((8,), -1e30, _F32)]).reshape(1, 16)

    @pl.when(i == 0)
    def _():
        m_o[...] = cur

    @pl.when(i > 0)
    def _():
        m_o[...] = jnp.maximum(m_o[...], cur)

    @pl.when(i == NG - 1)
    def _():
        acc = m_o[...]
        mx = acc[0, :4] + acc[0, 4:8]
        M = jnp.maximum(mx, 0.2 * mx)
        m_o[...] = jnp.tile(M, 4).reshape(1, 16)


def _k3(traw, y, dinv, wg, bg, wb, bb, xin, st, W, a_s, a_d):
    blk128 = pl.BlockSpec((BR, C), lambda i: (i, 0))
    return pl.pallas_call(
        _k3_body,
        grid=(NG,),
        in_specs=[
            blk128, blk128, blk128,
            pl.BlockSpec((BR, 1), lambda i: (i, 0)),
            pl.BlockSpec((HID, C), lambda i: (0, 0)),
            pl.BlockSpec((C,), lambda i: (0,)),
            pl.BlockSpec((HID, C), lambda i: (0, 0)),
            pl.BlockSpec((C,), lambda i: (0,)),
            blk128,
            pl.BlockSpec((2, 128), lambda i: (0, 0)),
            pl.BlockSpec((C, C), lambda i: (0, 0)),
            pl.BlockSpec((H, CPH), lambda i: (0, 0)),
            pl.BlockSpec((H, CPH), lambda i: (0, 0)),
        ],
        out_specs=[
            blk128,
            pl.BlockSpec((BR, 16), lambda i: (i, 0)),
            pl.BlockSpec((BR, 16), lambda i: (i, 0)),
            pl.BlockSpec((1, 16), lambda i: (0, 0)),
        ],
        out_shape=[
            jax.ShapeDtypeStruct((N, C), _F32),
            jax.ShapeDtypeStruct((N, 16), _F32),
            jax.ShapeDtypeStruct((N, 16), _F32),
            jax.ShapeDtypeStruct((1, 16), _F32),
        ],
    )(traw[:N], traw[NP:NP + N], y, dinv, wg, bg, wb, bb, xin, st, W, a_s, a_d)


def _k4_body(a0, a1, d0, d1, h_r, es_r, ed_r, m_r, b_r, add_r, out_o, st_o):
    i = pl.program_id(0)
    e = es_r[...] + ed_r[...]
    e = jnp.maximum(e, 0.2 * e)
    exl = jnp.exp(e - m_r[...])
    den = d0[...] + d1[...] + exl
    h = h_r[...]
    num = a0[...] + a1[...] + (h.reshape(BR, H, CPH)
                               * exl[:, :H, None]).reshape(BR, C)
    out = (num.reshape(BR, H, CPH) / den[:, :H, None]).reshape(BR, C)
    out = out + b_r[...] + add_r[...]
    out_o[...] = out
    cur = jnp.concatenate([jnp.sum(out, axis=0, keepdims=True),
                           jnp.sum(out * out, axis=0, keepdims=True)], axis=0)

    @pl.when(i == 0)
    def _():
        st_o[...] = cur

    @pl.when(i > 0)
    def _():
        st_o[...] = st_o[...] + cur

    @pl.when(i == NG - 1)
    def _():
        acc = st_o[...]
        mu = acc[0:1, :] / N
        var = acc[1:2, :] / N - mu * mu
        st_o[...] = jnp.concatenate([mu, lax.rsqrt(var + EPS)], axis=0)


def _k4(acc, den, h, es, ed, m16, b, add):
    blk128 = pl.BlockSpec((BR, C), lambda i: (i, 0))
    blk16 = pl.BlockSpec((BR, 16), lambda i: (i, 0))
    return pl.pallas_call(
        _k4_body,
        grid=(NG,),
        in_specs=[
            blk128, blk128, blk16, blk16, blk128, blk16, blk16,
            pl.BlockSpec((1, 16), lambda i: (0, 0)),
            pl.BlockSpec((C,), lambda i: (0,)),
            blk128,
        ],
        out_specs=[
            blk128,
            pl.BlockSpec((2, 128), lambda i: (0, 0)),
        ],
        out_shape=[
            jax.ShapeDtypeStruct((N, C), _F32),
            jax.ShapeDtypeStruct((2, 128), _F32),
        ],
    )(acc[:N], acc[NP:NP + N], den[:N], den[NP:NP + N], h, es, ed, m16, b, add)


@jax.jit
def kernel(x, mask, edge_index,
           s1_w0, s1_b0, s1_wg, s1_bg, s1_wb, s1_bb,
           s2_w0, s2_b0, s2_wg, s2_bg, s2_wb, s2_bb,
           ss_w0, ss_b0, ss_wg, ss_bg, ss_wb, ss_bb,
           g1_w, g1_as, g1_ad, g1_b,
           g2_w, g2_as, g2_ad, g2_b,
           gs_w, gs_as, gs_ad, gs_b):
    src = edge_index[0].astype(jnp.int32)
    dst = edge_index[1].astype(jnp.int32)
    z16 = jnp.zeros((NP, 16), _F32)
    z128 = jnp.zeros((NP, 128), _F32)
    ones16 = jnp.ones((CH, 16), _F32)
    zN = jnp.zeros((N, C), _F32)

    degacc = _sc_deg(dst, z16, ones16)
    dinv, ym, xst = _k1(degacc, mask, x)
    (sraw,) = _sc_seg([ym], src, dst, z16, 16)
    w0s = jnp.stack([s1_w0, s2_w0, ss_w0])
    b0s = jnp.stack([s1_b0, s2_b0, ss_b0])
    Y = _k2(sraw, ym, dinv, w0s, b0s)
    t1, t2, ts = _sc_seg([Y[0], Y[1], Y[2]], src, dst, z128, 128)

    h1, es1, ed1, m1 = _k3(t1, Y[0], dinv, s1_wg, s1_bg, s1_wb, s1_bb,
                           x, xst, g1_w, g1_as, g1_ad)
    hs, ess, eds, ms = _k3(ts, Y[2], dinv, ss_wg, ss_bg, ss_wb, ss_bb,
                           x, xst, gs_w, gs_as, gs_ad)
    acc1, den1, accs, dens = _sc_gat(
        [(h1, es1, ed1, m1), (hs, ess, eds, ms)], src, dst, z128, z16)
    x1, x1st = _k4(acc1, den1, h1, es1, ed1, m1, g1_b, zN)
    xs, _ = _k4(accs, dens, hs, ess, eds, ms, gs_b, zN)

    h2, es2, ed2, m2 = _k3(t2, Y[1], dinv, s2_wg, s2_bg, s2_wb, s2_bb,
                           x1, x1st, g2_w, g2_as, g2_ad)
    acc2, den2 = _sc_gat([(h2, es2, ed2, m2)], src, dst, z128, z16)
    out, _ = _k4(acc2, den2, h2, es2, ed2, m2, g2_b, xs)
    return out


# speedup vs baseline: 79.2873x; 1.1375x over previous
import functools
import jax
import jax.numpy as jnp
from jax import lax
from jax.experimental import pallas as pl
from jax.experimental.pallas import tpu as pltpu
from jax.experimental.pallas import tpu_sc as plsc

N = 10000
E = 320000
C = 128
CM = 16
HID = 128
H = 4
CPH = 32
EPS = 1e-05

NC = 2
NS = 16
NW = NC * NS
EPW = E // NW
CH = 100
NCH = EPW // CH
NP = 10240
RP = NP // NS
BR = 2000
NG = N // BR

_MESH = dict(
    mesh=plsc.VectorSubcoreMesh(core_axis_name="c", subcore_axis_name="s"),
    compiler_params=pltpu.CompilerParams(use_tc_tiling_on_sc=False,
                                         needs_layout_passes=False),
)
_F32 = jnp.float32


def _wid():
    return lax.axis_index("s") * NC + lax.axis_index("c")


def _sid():
    return lax.axis_index("s")


def _sc_deg(dst3, z16, ones16):
    def body(dst_h, z_h, ones_h, out_h, idx_a, idx_b, ones_v, acc_sh):
        c = lax.axis_index("c")
        s = _sid()
        wid = _wid()
        ibufs = (idx_a, idx_b)
        pltpu.sync_copy(z_h.at[pl.ds(s * RP, RP)], acc_sh.at[pl.ds(s * RP, RP)])
        pltpu.sync_copy(ones_h, ones_v)
        pltpu.sync_copy(dst_h.at[wid, 0], ibufs[0])
        plsc.subcore_barrier()

        def chunk2(jj, carry):
            for b in range(2):
                j = jj * 2 + b

                @pl.when(j + 1 < NCH)
                def _():
                    pltpu.sync_copy(dst_h.at[wid, j + 1], ibufs[1 - b])

                pltpu.sync_copy(ones_v, acc_sh.at[ibufs[b]], add=True)
            return carry

        lax.fori_loop(0, NCH // 2, chunk2, 0)
        plsc.subcore_barrier()
        pltpu.sync_copy(acc_sh.at[pl.ds(s * RP, RP)],
                        out_h.at[pl.ds(c * NP + s * RP, RP)])

    f = pl.kernel(
        body,
        out_type=jax.ShapeDtypeStruct((NC * NP, 16), _F32),
        scratch_types=[
            pltpu.VMEM((CH,), jnp.int32),
            pltpu.VMEM((CH,), jnp.int32),
            pltpu.VMEM((CH, 16), _F32),
            pltpu.VMEM_SHARED((NP, 16), _F32),
        ],
        **_MESH,
    )
    return f(dst3, z16, ones16)


def _sc_seg(tabs, src3, dst3, zk, k):
    P = len(tabs)

    def body(*refs):
        tab_h = refs[:P]
        src_h, dst_h, z_h = refs[P:P + 3]
        out_h = refs[P + 3:P + 3 + P]
        (ixs_a, ixs_b, ixd_a, ixd_b, rows_a, rows_b,
         acc_sh, sem_a, sem_b) = refs[P + 3 + P:]
        c = lax.axis_index("c")
        s = _sid()
        wid = _wid()
        sbufs = (ixs_a, ixs_b)
        dbufs = (ixd_a, ixd_b)
        rbufs = (rows_a, rows_b)
        sems = (sem_a, sem_b)
        for p in range(P):
            pltpu.sync_copy(z_h.at[pl.ds(s * RP, RP)], acc_sh.at[pl.ds(s * RP, RP)])
            plsc.subcore_barrier()
            pltpu.sync_copy(src_h.at[wid, 0], sbufs[0])
            pltpu.sync_copy(dst_h.at[wid, 0], dbufs[0])
            pltpu.async_copy(tab_h[p].at[sbufs[0]], rbufs[0], sems[0])

            def chunk2(jj, carry, p=p):
                for b in range(2):
                    j = jj * 2 + b

                    @pl.when(j + 1 < NCH)
                    def _(b=b, j=j, p=p):
                        pltpu.sync_copy(src_h.at[wid, j + 1], sbufs[1 - b])
                        pltpu.sync_copy(dst_h.at[wid, j + 1], dbufs[1 - b])
                        pltpu.async_copy(tab_h[p].at[sbufs[1 - b]],
                                         rbufs[1 - b], sems[1 - b])

                    pltpu.make_async_copy(tab_h[p].at[sbufs[b]],
                                          rbufs[b], sems[b]).wait()
                    pltpu.sync_copy(rbufs[b], acc_sh.at[dbufs[b]], add=True)
                return carry

            lax.fori_loop(0, NCH // 2, chunk2, 0)
            plsc.subcore_barrier()
            pltpu.sync_copy(acc_sh.at[pl.ds(s * RP, RP)],
                            out_h[p].at[pl.ds(c * NP + s * RP, RP)])
            plsc.subcore_barrier()

    f = pl.kernel(
        body,
        out_type=[jax.ShapeDtypeStruct((NC * NP, k), _F32) for _ in range(P)],
        scratch_types=[
            pltpu.VMEM((CH,), jnp.int32),
            pltpu.VMEM((CH,), jnp.int32),
            pltpu.VMEM((CH,), jnp.int32),
            pltpu.VMEM((CH,), jnp.int32),
            pltpu.VMEM((CH, k), _F32),
            pltpu.VMEM((CH, k), _F32),
            pltpu.VMEM_SHARED((NP, k), _F32),
            pltpu.SemaphoreType.DMA,
            pltpu.SemaphoreType.DMA,
        ],
        **_MESH,
    )
    return f(*tabs, src3, dst3, zk)


def _sc_gat(gats, src3, dst3, z128, z16):
    G = len(gats)

    def body(*refs):
        ins = refs[:4 * G]
        src_h, dst_h, z128_h, z16_h = refs[4 * G:4 * G + 4]
        outs = refs[4 * G + 4:4 * G + 4 + 2 * G]
        (ixs_a, ixs_b, ixd_a, ixd_b, rows_a, rows_b, esr_a, esr_b,
         edr_a, edr_b, exb, m_v, acc_sh, den_sh, sem_a, sem_b) = \
            refs[4 * G + 4 + 2 * G:]
        c = lax.axis_index("c")
        s = _sid()
        wid = _wid()
        isb = (ixs_a, ixs_b)
        idb = (ixd_a, ixd_b)
        rbufs = (rows_a, rows_b)
        sbufs = (esr_a, esr_b)
        dbufs = (edr_a, edr_b)
        sems = (sem_a, sem_b)

        def issue(h_h, es_h, ed_h, j, b):
            pltpu.sync_copy(src_h.at[wid, j], isb[b])
            pltpu.sync_copy(dst_h.at[wid, j], idb[b])
            pltpu.async_copy(h_h.at[isb[b]], rbufs[b], sems[b])
            pltpu.async_copy(es_h.at[isb[b]], sbufs[b], sems[b])
            pltpu.async_copy(ed_h.at[idb[b]], dbufs[b], sems[b])

        def drain(h_h, es_h, ed_h, b):
            pltpu.make_async_copy(h_h.at[isb[b]], rbufs[b], sems[b]).wait()
            pltpu.make_async_copy(es_h.at[isb[b]], sbufs[b], sems[b]).wait()
            pltpu.make_async_copy(ed_h.at[idb[b]], dbufs[b], sems[b]).wait()

        for g in range(G):
            h_h, es_h, ed_h, m_h = ins[4 * g:4 * g + 4]
            acc_o, den_o = outs[2 * g:2 * g + 2]
            pltpu.sync_copy(z128_h.at[pl.ds(s * RP, RP)],
                            acc_sh.at[pl.ds(s * RP, RP)])
            pltpu.sync_copy(z16_h.at[pl.ds(s * RP, RP)],
                            den_sh.at[pl.ds(s * RP, RP)])
            pltpu.sync_copy(m_h, m_v)
            plsc.subcore_barrier()
            mv = m_v[0, :]
            issue(h_h, es_h, ed_h, 0, 0)

            def chunk2(jj, carry, h_h=h_h, es_h=es_h, ed_h=ed_h, mv=mv):
                for b in range(2):
                    j = jj * 2 + b

                    @pl.when(j + 1 < NCH)
                    def _(b=b, j=j):
                        issue(h_h, es_h, ed_h, j + 1, 1 - b)

                    drain(h_h, es_h, ed_h, b)
                    rows_v = rbufs[b]
                    esr = sbufs[b]
                    edr = dbufs[b]

                    @plsc.parallel_loop(0, CH, unroll=4)
                    def row(r, rows_v=rows_v, esr=esr, edr=edr, mv=mv):
                        e16 = esr[r, :] + edr[r, :]
                        e16 = jnp.maximum(e16, e16 * 0.2)
                        ex16 = jnp.exp(e16 - mv)
                        exb[r, :] = ex16
                        for hh in range(4):
                            spl = ex16.at[jnp.full((16,), hh, jnp.int32)].get(
                                mode="promise_in_bounds")
                            rows_v[r, pl.ds(hh * 32, 16)] = \
                                rows_v[r, pl.ds(hh * 32, 16)] * spl
                            rows_v[r, pl.ds(hh * 32 + 16, 16)] = \
                                rows_v[r, pl.ds(hh * 32 + 16, 16)] * spl

                    pltpu.sync_copy(rows_v, acc_sh.at[idb[b]], add=True)
                    pltpu.sync_copy(exb, den_sh.at[idb[b]], add=True)
                return carry

            lax.fori_loop(0, NCH // 2, chunk2, 0)
            plsc.subcore_barrier()
            pltpu.sync_copy(acc_sh.at[pl.ds(s * RP, RP)],
                            acc_o.at[pl.ds(c * NP + s * RP, RP)])
            pltpu.sync_copy(den_sh.at[pl.ds(s * RP, RP)],
                            den_o.at[pl.ds(c * NP + s * RP, RP)])
            plsc.subcore_barrier()

    out_type = []
    for _ in range(G):
        out_type.append(jax.ShapeDtypeStruct((NC * NP, 128), _F32))
        out_type.append(jax.ShapeDtypeStruct((NC * NP, 16), _F32))
    f = pl.kernel(
        body,
        out_type=out_type,
        scratch_types=[
            pltpu.VMEM((CH,), jnp.int32),
            pltpu.VMEM((CH,), jnp.int32),
            pltpu.VMEM((CH,), jnp.int32),
            pltpu.VMEM((CH,), jnp.int32),
            pltpu.VMEM((CH, 128), _F32),
            pltpu.VMEM((CH, 128), _F32),
            pltpu.VMEM((CH, 16), _F32),
            pltpu.VMEM((CH, 16), _F32),
            pltpu.VMEM((CH, 16), _F32),
            pltpu.VMEM((CH, 16), _F32),
            pltpu.VMEM((CH, 16), _F32),
            pltpu.VMEM((1, 16), _F32),
            pltpu.VMEM_SHARED((NP, 128), _F32),
            pltpu.VMEM_SHARED((NP, 16), _F32),
            pltpu.SemaphoreType.DMA,
            pltpu.SemaphoreType.DMA,
        ],
        **_MESH,
    )
    flat = []
    for g in gats:
        flat.extend(g)
    return f(*flat, src3, dst3, z128, z16)


def _k1_body(d0, d1, mask_r, x_r, dinv_o, ym_o, st_o):
    deg = d0[:, 0:1] + d1[:, 0:1] + 1.0
    dinv = lax.rsqrt(deg)
    dinv_o[...] = dinv
    ym_o[...] = mask_r[...] * dinv
    xv = x_r[...]
    mu = jnp.mean(xv, axis=0, keepdims=True)
    var = jnp.mean(xv * xv, axis=0, keepdims=True) - mu * mu
    st_o[...] = jnp.concatenate([mu, lax.rsqrt(var + EPS)], axis=0)


def _k1(degacc, mask, x):
    return pl.pallas_call(
        _k1_body,
        out_shape=[
            jax.ShapeDtypeStruct((N, 1), _F32),
            jax.ShapeDtypeStruct((N, 16), _F32),
            jax.ShapeDtypeStruct((2, 128), _F32),
        ],
    )(degacc[:N], degacc[NP:NP + N], mask, x)


def _k2_body(s0, s1, ym_r, dinv_r, w_r, b_r, y_o):
    dinv = dinv_r[...]
    S = dinv * (s0[...] + s1[...] + ym_r[...])
    for p in range(3):
        mf = jnp.dot(S, w_r[p], preferred_element_type=_F32) + b_r[p]
        mf = jnp.maximum(mf, 0.0)
        y_o[p] = mf * dinv


def _k2(sraw, ym, dinv, w0s, b0s):
    return pl.pallas_call(
        _k2_body,
        grid=(NG,),
        in_specs=[
            pl.BlockSpec((BR, 16), lambda i: (i, 0)),
            pl.BlockSpec((BR, 16), lambda i: (i, 0)),
            pl.BlockSpec((BR, 16), lambda i: (i, 0)),
            pl.BlockSpec((BR, 1), lambda i: (i, 0)),
            pl.BlockSpec((3, CM, HID), lambda i: (0, 0, 0)),
            pl.BlockSpec((3, HID), lambda i: (0, 0)),
        ],
        out_specs=pl.BlockSpec((3, BR, HID), lambda i: (0, i, 0)),
        out_shape=jax.ShapeDtypeStruct((3, N, HID), _F32),
    )(sraw[:N], sraw[NP:NP + N], ym, dinv, w0s, b0s)


def _k3_body(t0, t1, y_r, dinv_r, wg, bg, wb, bb, xin, st, W, a_s, a_d,
             h_o, es_o, ed_o, m_o):
    i = pl.program_id(0)
    dinv = dinv_r[...]
    T = dinv * (t0[...] + t1[...] + y_r[...])
    gamma = jnp.dot(T, wg[...], preferred_element_type=_F32) + bg[...]
    beta = jnp.dot(T, wb[...], preferred_element_type=_F32) + bb[...]
    xb = (xin[...] - st[0:1, :]) * st[1:2, :]
    xa = xb * (1.0 + gamma) + beta
    xa = jnp.maximum(xa, 0.2 * xa)
    h = jnp.dot(xa, W[...], preferred_element_type=_F32)
    h_o[...] = h
    h4 = h.reshape(BR, H, CPH)
    es = jnp.sum(h4 * a_s[...][None], axis=2)
    ed = jnp.sum(h4 * a_d[...][None], axis=2)
    zpad = jnp.zeros((BR, 12), _F32)
    es_o[...] = jnp.concatenate([es, zpad], axis=1)
    ed_o[...] = jnp.concatenate([ed, zpad], axis=1)
    cur = jnp.concatenate(
        [jnp.max(es, axis=0), jnp.max(ed, axis=0),
         jnp.full((8,), -1e30, _F32)]).reshape(1, 16)

    @pl.when(i == 0)
    def _():
        m_o[...] = cur

    @pl.when(i > 0)
    def _():
        m_o[...] = jnp.maximum(m_o[...], cur)

    @pl.when(i == NG - 1)
    def _():
        acc = m_o[...]
        mx = acc[0, :4] + acc[0, 4:8]
        M = jnp.maximum(mx, 0.2 * mx)
        m_o[...] = jnp.tile(M, 4).reshape(1, 16)


def _k3(traw, y, dinv, wg, bg, wb, bb, xin, st, W, a_s, a_d):
    blk128 = pl.BlockSpec((BR, C), lambda i: (i, 0))
    return pl.pallas_call(
        _k3_body,
        grid=(NG,),
        in_specs=[
            blk128, blk128, blk128,
            pl.BlockSpec((BR, 1), lambda i: (i, 0)),
            pl.BlockSpec((HID, C), lambda i: (0, 0)),
            pl.BlockSpec((C,), lambda i: (0,)),
            pl.BlockSpec((HID, C), lambda i: (0, 0)),
            pl.BlockSpec((C,), lambda i: (0,)),
            blk128,
            pl.BlockSpec((2, 128), lambda i: (0, 0)),
            pl.BlockSpec((C, C), lambda i: (0, 0)),
            pl.BlockSpec((H, CPH), lambda i: (0, 0)),
            pl.BlockSpec((H, CPH), lambda i: (0, 0)),
        ],
        out_specs=[
            blk128,
            pl.BlockSpec((BR, 16), lambda i: (i, 0)),
            pl.BlockSpec((BR, 16), lambda i: (i, 0)),
            pl.BlockSpec((1, 16), lambda i: (0, 0)),
        ],
        out_shape=[
            jax.ShapeDtypeStruct((N, C), _F32),
            jax.ShapeDtypeStruct((N, 16), _F32),
            jax.ShapeDtypeStruct((N, 16), _F32),
            jax.ShapeDtypeStruct((1, 16), _F32),
        ],
    )(traw[:N], traw[NP:NP + N], y, dinv, wg, bg, wb, bb, xin, st, W, a_s, a_d)


def _k4_body(a0, a1, d0, d1, h_r, es_r, ed_r, m_r, b_r, add_r, out_o, st_o):
    i = pl.program_id(0)
    e = es_r[...] + ed_r[...]
    e = jnp.maximum(e, 0.2 * e)
    exl = jnp.exp(e - m_r[...])
    den = d0[...] + d1[...] + exl
    h = h_r[...]
    num = a0[...] + a1[...] + (h.reshape(BR, H, CPH)
                               * exl[:, :H, None]).reshape(BR, C)
    out = (num.reshape(BR, H, CPH) / den[:, :H, None]).reshape(BR, C)
    out = out + b_r[...] + add_r[...]
    out_o[...] = out
    cur = jnp.concatenate([jnp.sum(out, axis=0, keepdims=True),
                           jnp.sum(out * out, axis=0, keepdims=True)], axis=0)

    @pl.when(i == 0)
    def _():
        st_o[...] = cur

    @pl.when(i > 0)
    def _():
        st_o[...] = st_o[...] + cur

    @pl.when(i == NG - 1)
    def _():
        acc = st_o[...]
        mu = acc[0:1, :] / N
        var = acc[1:2, :] / N - mu * mu
        st_o[...] = jnp.concatenate([mu, lax.rsqrt(var + EPS)], axis=0)


def _k4(acc, den, h, es, ed, m16, b, add):
    blk128 = pl.BlockSpec((BR, C), lambda i: (i, 0))
    blk16 = pl.BlockSpec((BR, 16), lambda i: (i, 0))
    return pl.pallas_call(
        _k4_body,
        grid=(NG,),
        in_specs=[
            blk128, blk128, blk16, blk16, blk128, blk16, blk16,
            pl.BlockSpec((1, 16), lambda i: (0, 0)),
            pl.BlockSpec((C,), lambda i: (0,)),
            blk128,
        ],
        out_specs=[
            blk128,
            pl.BlockSpec((2, 128), lambda i: (0, 0)),
        ],
        out_shape=[
            jax.ShapeDtypeStruct((N, C), _F32),
            jax.ShapeDtypeStruct((2, 128), _F32),
        ],
    )(acc[:N], acc[NP:NP + N], den[:N], den[NP:NP + N], h, es, ed, m16, b, add)


@jax.jit
def kernel(x, mask, edge_index,
           s1_w0, s1_b0, s1_wg, s1_bg, s1_wb, s1_bb,
           s2_w0, s2_b0, s2_wg, s2_bg, s2_wb, s2_bb,
           ss_w0, ss_b0, ss_wg, ss_bg, ss_wb, ss_bb,
           g1_w, g1_as, g1_ad, g1_b,
           g2_w, g2_as, g2_ad, g2_b,
           gs_w, gs_as, gs_ad, gs_b):
    src = edge_index[0].astype(jnp.int32).reshape(NW, NCH, CH)
    dst = edge_index[1].astype(jnp.int32).reshape(NW, NCH, CH)
    z16 = jnp.zeros((NP, 16), _F32)
    z128 = jnp.zeros((NP, 128), _F32)
    ones16 = jnp.ones((CH, 16), _F32)
    zN = jnp.zeros((N, C), _F32)

    degacc = _sc_deg(dst, z16, ones16)
    dinv, ym, xst = _k1(degacc, mask, x)
    (sraw,) = _sc_seg([ym], src, dst, z16, 16)
    w0s = jnp.stack([s1_w0, s2_w0, ss_w0])
    b0s = jnp.stack([s1_b0, s2_b0, ss_b0])
    Y = _k2(sraw, ym, dinv, w0s, b0s)
    t1, t2, ts = _sc_seg([Y[0], Y[1], Y[2]], src, dst, z128, 128)

    h1, es1, ed1, m1 = _k3(t1, Y[0], dinv, s1_wg, s1_bg, s1_wb, s1_bb,
                           x, xst, g1_w, g1_as, g1_ad)
    hs, ess, eds, ms = _k3(ts, Y[2], dinv, ss_wg, ss_bg, ss_wb, ss_bb,
                           x, xst, gs_w, gs_as, gs_ad)
    acc1, den1, accs, dens = _sc_gat(
        [(h1, es1, ed1, m1), (hs, ess, eds, ms)], src, dst, z128, z16)
    x1, x1st = _k4(acc1, den1, h1, es1, ed1, m1, g1_b, zN)
    xs, _ = _k4(accs, dens, hs, ess, eds, ms, gs_b, zN)

    h2, es2, ed2, m2 = _k3(t2, Y[1], dinv, s2_wg, s2_bg, s2_wb, s2_bb,
                           x1, x1st, g2_w, g2_as, g2_ad)
    acc2, den2 = _sc_gat([(h2, es2, ed2, m2)], src, dst, z128, z16)
    out, _ = _k4(acc2, den2, h2, es2, ed2, m2, g2_b, xs)
    return out
